# baseline (speedup 1.0000x reference)
import jax
import jax.numpy as jnp
from jax.experimental import pallas as pl
from jax.experimental.pallas import tpu as pltpu

EPS = 1e-06

_B, _S = 4096, 2048
_BB = 256
_NB = _B // _BB
_CT = _S // 128
_NT = (_BB // 8) * _CT


def _body(yp_ref, lab_ref, ctl_ref, mf_ref, mc_ref, out_ref, acc_ref):
    i = pl.program_id(0)

    @pl.when(i == 0)
    def _init():
        acc_ref[...] = jnp.zeros_like(acc_ref)

    zero = jnp.zeros((8, 128), jnp.float32)

    def step(j, accs):
        (n1, sp1, st1, spt1, spp1, stt1,
         n2, sp2, st2, spt2, spp2, stt2, sabs,
         n3, sp3, st3, spt3, spp3, stt3) = accs
        r = (j // _CT) * 8
        c = (j % _CT) * 128
        p0 = yp_ref[pl.ds(r, 8), 0, pl.ds(c, 128)]
        p1 = yp_ref[pl.ds(r, 8), 1, pl.ds(c, 128)]
        t = lab_ref[pl.ds(r, 8), pl.ds(c, 128)]
        tc = ctl_ref[pl.ds(r, 8), pl.ds(c, 128)]
        mf = mf_ref[pl.ds(r, 8), pl.ds(c, 128)].astype(jnp.float32)
        mc = mc_ref[pl.ds(r, 8), pl.ds(c, 128)].astype(jnp.float32)
        md = mf * mc
        full = p0 + p1
        diff = t - tc
        u1 = p0 * mc
        v1 = tc * mc
        u2 = full * mf
        v2 = t * mf
        u3 = p1 * md
        v3 = diff * md
        return (n1 + mc, sp1 + u1, st1 + v1,
                spt1 + u1 * v1, spp1 + u1 * u1, stt1 + v1 * v1,
                n2 + mf, sp2 + u2, st2 + v2,
                spt2 + u2 * v2, spp2 + u2 * u2, stt2 + v2 * v2,
                sabs + jnp.abs(u2 - v2),
                n3 + md, sp3 + u3, st3 + v3,
                spt3 + u3 * v3, spp3 + u3 * u3, stt3 + v3 * v3)

    accs = jax.lax.fori_loop(0, _NT, step, (zero,) * 19)
    for k in range(19):
        acc_ref[k] += accs[k]

    @pl.when(i == _NB - 1)
    def _fin():
        def corr(base):
            n = jnp.sum(acc_ref[base + 0])
            sp = jnp.sum(acc_ref[base + 1])
            st = jnp.sum(acc_ref[base + 2])
            spt = jnp.sum(acc_ref[base + 3])
            spp = jnp.sum(acc_ref[base + 4])
            stt = jnp.sum(acc_ref[base + 5])
            dot = spt - sp * st / n
            na = jnp.sqrt(spp - sp * sp / n)
            nb = jnp.sqrt(stt - st * st / n)
            return dot / (jnp.maximum(na, EPS) * jnp.maximum(nb, EPS)), n

        corr_ctrl, _ = corr(0)
        corr_full, n2 = corr(6)
        corr_diff, _ = corr(13)
        l1 = jnp.sqrt(jnp.sum(acc_ref[12]) / n2)
        out_ref[0] = 1.0 - corr_ctrl
        out_ref[1] = (1.0 - corr_full) + l1
        out_ref[2] = corr_full
        out_ref[3] = l1
        out_ref[4] = 1.0 - corr_diff


@jax.jit
def _reduce(y_pred, labels, labels_ctrl, mask_full, mask_ctrl):
    return pl.pallas_call(
        _body,
        grid=(_NB,),
        in_specs=[
            pl.BlockSpec((_BB, 2, _S), lambda i: (i, 0, 0)),
            pl.BlockSpec((_BB, _S), lambda i: (i, 0)),
            pl.BlockSpec((_BB, _S), lambda i: (i, 0)),
            pl.BlockSpec((_BB, _S), lambda i: (i, 0)),
            pl.BlockSpec((_BB, _S), lambda i: (i, 0)),
        ],
        out_specs=pl.BlockSpec(memory_space=pltpu.SMEM),
        out_shape=jax.ShapeDtypeStruct((8,), jnp.float32),
        scratch_shapes=[pltpu.VMEM((19, 8, 128), jnp.float32)],
    )(y_pred, labels, labels_ctrl, mask_full, mask_ctrl)


def kernel(y_pred, labels, labels_ctrl, mask_full, mask_ctrl, condition_):
    out = _reduce(y_pred, labels, labels_ctrl, mask_full, mask_ctrl)
    loss_ctrl, loss_full, perf, l1, loss_depr = (
        out[0], out[1], out[2], out[3], out[4])
    loss = jnp.where(condition_ != 64,
                     loss_ctrl + loss_depr + loss_full,
                     loss_ctrl + loss_full)
    return (loss, perf, l1)


# speedup vs baseline: 1.5281x; 1.5281x over previous
import jax
import jax.numpy as jnp
from jax.experimental import pallas as pl
from jax.experimental.pallas import tpu as pltpu

EPS = 1e-06

_B, _S = 4096, 2048
_BB = 256
_NB = _B // _BB
_CT = _S // 128
_NT = (_BB // 8) * _CT


def _body(yp_ref, lab_ref, ctl_ref, mf_ref, mc_ref, out_ref, acc_ref):
    i = pl.program_id(0)

    @pl.when(i == 0)
    def _init():
        acc_ref[...] = jnp.zeros_like(acc_ref)

    zero = jnp.zeros((8, 128), jnp.float32)

    def step(j, accs):
        (n1, sp1, st1, spt1, spp1, stt1,
         n2, sp2, st2, spt2, spp2, stt2, sabs,
         n3, sp3, st3, spt3, spp3, stt3) = accs
        r = (j // _CT) * 8
        c = (j % _CT) * 128
        p0 = yp_ref[pl.ds(r, 8), 0, pl.ds(c, 128)]
        p1 = yp_ref[pl.ds(r, 8), 1, pl.ds(c, 128)]
        t = lab_ref[pl.ds(r, 8), pl.ds(c, 128)]
        tc = ctl_ref[pl.ds(r, 8), pl.ds(c, 128)]
        mf = mf_ref[pl.ds(r, 8), pl.ds(c, 128)].astype(jnp.float32)
        mc = mc_ref[pl.ds(r, 8), pl.ds(c, 128)].astype(jnp.float32)
        md = mf * mc
        full = p0 + p1
        diff = t - tc
        u1 = p0 * mc
        v1 = tc * mc
        u2 = full * mf
        v2 = t * mf
        u3 = p1 * md
        v3 = diff * md
        return (n1 + mc, sp1 + u1, st1 + v1,
                spt1 + u1 * v1, spp1 + u1 * u1, stt1 + v1 * v1,
                n2 + mf, sp2 + u2, st2 + v2,
                spt2 + u2 * v2, spp2 + u2 * u2, stt2 + v2 * v2,
                sabs + jnp.abs(u2 - v2),
                n3 + md, sp3 + u3, st3 + v3,
                spt3 + u3 * v3, spp3 + u3 * u3, stt3 + v3 * v3)

    accs = jax.lax.fori_loop(0, _NT, step, (zero,) * 19, unroll=8)
    for k in range(19):
        acc_ref[k] += accs[k]

    @pl.when(i == _NB - 1)
    def _fin():
        def corr(base):
            n = jnp.sum(acc_ref[base + 0])
            sp = jnp.sum(acc_ref[base + 1])
            st = jnp.sum(acc_ref[base + 2])
            spt = jnp.sum(acc_ref[base + 3])
            spp = jnp.sum(acc_ref[base + 4])
            stt = jnp.sum(acc_ref[base + 5])
            dot = spt - sp * st / n
            na = jnp.sqrt(spp - sp * sp / n)
            nb = jnp.sqrt(stt - st * st / n)
            return dot / (jnp.maximum(na, EPS) * jnp.maximum(nb, EPS)), n

        corr_ctrl, _ = corr(0)
        corr_full, n2 = corr(6)
        corr_diff, _ = corr(13)
        l1 = jnp.sqrt(jnp.sum(acc_ref[12]) / n2)
        out_ref[0] = 1.0 - corr_ctrl
        out_ref[1] = (1.0 - corr_full) + l1
        out_ref[2] = corr_full
        out_ref[3] = l1
        out_ref[4] = 1.0 - corr_diff


@jax.jit
def _reduce(y_pred, labels, labels_ctrl, mask_full, mask_ctrl):
    return pl.pallas_call(
        _body,
        grid=(_NB,),
        in_specs=[
            pl.BlockSpec((_BB, 2, _S), lambda i: (i, 0, 0)),
            pl.BlockSpec((_BB, _S), lambda i: (i, 0)),
            pl.BlockSpec((_BB, _S), lambda i: (i, 0)),
            pl.BlockSpec((_BB, _S), lambda i: (i, 0)),
            pl.BlockSpec((_BB, _S), lambda i: (i, 0)),
        ],
        out_specs=pl.BlockSpec(memory_space=pltpu.SMEM),
        out_shape=jax.ShapeDtypeStruct((8,), jnp.float32),
        scratch_shapes=[pltpu.VMEM((19, 8, 128), jnp.float32)],
    )(y_pred, labels, labels_ctrl, mask_full, mask_ctrl)


def kernel(y_pred, labels, labels_ctrl, mask_full, mask_ctrl, condition_):
    out = _reduce(y_pred, labels, labels_ctrl, mask_full, mask_ctrl)
    loss_ctrl, loss_full, perf, l1, loss_depr = (
        out[0], out[1], out[2], out[3], out[4])
    loss = jnp.where(condition_ != 64,
                     loss_ctrl + loss_depr + loss_full,
                     loss_ctrl + loss_full)
    return (loss, perf, l1)


# speedup vs baseline: 2.9902x; 1.9568x over previous
import jax
import jax.numpy as jnp
from jax.experimental import pallas as pl
from jax.experimental.pallas import tpu as pltpu

EPS = 1e-06

_B, _S = 4096, 2048
_BB = 128
_NB = _B // _BB


def _body(yp_ref, lab_ref, ctl_ref, mf_ref, mc_ref, out_ref, acc_ref):
    i = pl.program_id(0)

    @pl.when(i == 0)
    def _init():
        acc_ref[...] = jnp.zeros_like(acc_ref)

    p0 = yp_ref[:, 0, :]
    p1 = yp_ref[:, 1, :]
    t = lab_ref[...]
    tc = ctl_ref[...]
    mf = mf_ref[...].astype(jnp.float32)
    mc = mc_ref[...].astype(jnp.float32)
    md = mf * mc

    full = p0 + p1
    diff = t - tc

    def fold(x):
        return jnp.sum(x.reshape(_BB // 8, 8, _S), axis=0)

    def sums(p, t_, m, base):
        u = p * m
        v = t_ * m
        acc_ref[base + 0] += fold(m)
        acc_ref[base + 1] += fold(u)
        acc_ref[base + 2] += fold(v)
        acc_ref[base + 3] += fold(u * v)
        acc_ref[base + 4] += fold(u * u)
        acc_ref[base + 5] += fold(v * v)
        return u, v

    sums(p0, tc, mc, 0)
    u2, v2 = sums(full, t, mf, 6)
    acc_ref[18] += fold(jnp.abs(u2 - v2))
    sums(p1, diff, md, 12)

    @pl.when(i == _NB - 1)
    def _fin():
        def corr(base):
            n = jnp.sum(acc_ref[base + 0])
            sp = jnp.sum(acc_ref[base + 1])
            st = jnp.sum(acc_ref[base + 2])
            spt = jnp.sum(acc_ref[base + 3])
            spp = jnp.sum(acc_ref[base + 4])
            stt = jnp.sum(acc_ref[base + 5])
            dot = spt - sp * st / n
            na = jnp.sqrt(spp - sp * sp / n)
            nb = jnp.sqrt(stt - st * st / n)
            return dot / (jnp.maximum(na, EPS) * jnp.maximum(nb, EPS)), n

        corr_ctrl, _ = corr(0)
        corr_full, n2 = corr(6)
        corr_diff, _ = corr(12)
        l1 = jnp.sqrt(jnp.sum(acc_ref[18]) / n2)
        out_ref[0] = 1.0 - corr_ctrl
        out_ref[1] = (1.0 - corr_full) + l1
        out_ref[2] = corr_full
        out_ref[3] = l1
        out_ref[4] = 1.0 - corr_diff


@jax.jit
def _reduce(y_pred, labels, labels_ctrl, mask_full, mask_ctrl):
    return pl.pallas_call(
        _body,
        grid=(_NB,),
        in_specs=[
            pl.BlockSpec((_BB, 2, _S), lambda i: (i, 0, 0)),
            pl.BlockSpec((_BB, _S), lambda i: (i, 0)),
            pl.BlockSpec((_BB, _S), lambda i: (i, 0)),
            pl.BlockSpec((_BB, _S), lambda i: (i, 0)),
            pl.BlockSpec((_BB, _S), lambda i: (i, 0)),
        ],
        out_specs=pl.BlockSpec(memory_space=pltpu.SMEM),
        out_shape=jax.ShapeDtypeStruct((8,), jnp.float32),
        scratch_shapes=[pltpu.VMEM((19, 8, _S), jnp.float32)],
    )(y_pred, labels, labels_ctrl, mask_full, mask_ctrl)


def kernel(y_pred, labels, labels_ctrl, mask_full, mask_ctrl, condition_):
    out = _reduce(y_pred, labels, labels_ctrl, mask_full, mask_ctrl)
    loss_ctrl, loss_full, perf, l1, loss_depr = (
        out[0], out[1], out[2], out[3], out[4])
    loss = jnp.where(condition_ != 64,
                     loss_ctrl + loss_depr + loss_full,
                     loss_ctrl + loss_full)
    return (loss, perf, l1)


# speedup vs baseline: 4.4965x; 1.5038x over previous
import jax
import jax.numpy as jnp
from jax.experimental import pallas as pl
from jax.experimental.pallas import tpu as pltpu

EPS = 1e-06

_B, _S = 4096, 2048
_BB = 256
_NB = _B // _BB


def _body(yp_ref, lab_ref, ctl_ref, mf_ref, mc_ref, out_ref, acc_ref):
    i = pl.program_id(0)

    @pl.when(i == 0)
    def _init():
        acc_ref[...] = jnp.zeros_like(acc_ref)

    p0 = yp_ref[:, 0, :]
    p1 = yp_ref[:, 1, :]
    t = lab_ref[...]
    tc = ctl_ref[...]
    mf = mf_ref[...]
    mc = mc_ref[...]
    md = mf & mc

    full = p0 + p1
    diff = t - tc

    def fold(x):
        return jnp.sum(x.reshape(_BB // 8, 8, _S), axis=0)

    def sums(p, t_, m, base):
        u = jnp.where(m, p, 0.0)
        v = jnp.where(m, t_, 0.0)
        acc_ref[base + 0] += fold(jnp.where(m, 1.0, 0.0))
        acc_ref[base + 1] += fold(u)
        acc_ref[base + 2] += fold(v)
        acc_ref[base + 3] += fold(u * v)
        acc_ref[base + 4] += fold(u * u)
        acc_ref[base + 5] += fold(v * v)
        return u, v

    sums(p0, tc, mc, 0)
    u2, v2 = sums(full, t, mf, 6)
    acc_ref[18] += fold(jnp.abs(u2 - v2))
    sums(p1, diff, md, 12)

    @pl.when(i == _NB - 1)
    def _fin():
        def corr(base):
            n = jnp.sum(acc_ref[base + 0])
            sp = jnp.sum(acc_ref[base + 1])
            st = jnp.sum(acc_ref[base + 2])
            spt = jnp.sum(acc_ref[base + 3])
            spp = jnp.sum(acc_ref[base + 4])
            stt = jnp.sum(acc_ref[base + 5])
            dot = spt - sp * st / n
            na = jnp.sqrt(spp - sp * sp / n)
            nb = jnp.sqrt(stt - st * st / n)
            return dot / (jnp.maximum(na, EPS) * jnp.maximum(nb, EPS)), n

        corr_ctrl, _ = corr(0)
        corr_full, n2 = corr(6)
        corr_diff, _ = corr(12)
        l1 = jnp.sqrt(jnp.sum(acc_ref[18]) / n2)
        out_ref[0] = 1.0 - corr_ctrl
        out_ref[1] = (1.0 - corr_full) + l1
        out_ref[2] = corr_full
        out_ref[3] = l1
        out_ref[4] = 1.0 - corr_diff


@jax.jit
def _reduce(y_pred, labels, labels_ctrl, mask_full, mask_ctrl):
    return pl.pallas_call(
        _body,
        grid=(_NB,),
        in_specs=[
            pl.BlockSpec((_BB, 2, _S), lambda i: (i, 0, 0)),
            pl.BlockSpec((_BB, _S), lambda i: (i, 0)),
            pl.BlockSpec((_BB, _S), lambda i: (i, 0)),
            pl.BlockSpec((_BB, _S), lambda i: (i, 0)),
            pl.BlockSpec((_BB, _S), lambda i: (i, 0)),
        ],
        out_specs=pl.BlockSpec(memory_space=pltpu.SMEM),
        out_shape=jax.ShapeDtypeStruct((8,), jnp.float32),
        scratch_shapes=[pltpu.VMEM((19, 8, _S), jnp.float32)],
    )(y_pred, labels, labels_ctrl, mask_full, mask_ctrl)


def kernel(y_pred, labels, labels_ctrl, mask_full, mask_ctrl, condition_):
    out = _reduce(y_pred, labels, labels_ctrl, mask_full, mask_ctrl)
    loss_ctrl, loss_full, perf, l1, loss_depr = (
        out[0], out[1], out[2], out[3], out[4])
    loss = jnp.where(condition_ != 64,
                     loss_ctrl + loss_depr + loss_full,
                     loss_ctrl + loss_full)
    return (loss, perf, l1)
